# Initial kernel scaffold; baseline (speedup 1.0000x reference)
import jax
import jax.numpy as jnp
from jax.experimental import pallas as pl


def kernel(x, edge_index, W1l, b1l, W1r, W2l, b2l, W2r):
    raise NotImplementedError("write your pallas kernel here")


# speedup vs baseline: 3.8898x; 3.8898x over previous
import functools

import jax
import jax.numpy as jnp
from jax import lax
from jax.experimental import pallas as pl
from jax.experimental.pallas import tpu as pltpu
from jax.experimental.pallas import tpu_sc as plsc

N_NODES = 10000
N_EDGES = 320000
D = 128

NC = 2
NS = 16
NW = NC * NS
C = 128
NCH = 79
E_PAD = NW * NCH * C
SLAB = 640
N_PAD = NS * SLAB


def _sc_agg_body(feat, src_r, dst_r, zrows, agg_out,
                 sidx, didx, rows, sem, agg_sh):
    cid = lax.axis_index("c")
    sid = lax.axis_index("s")
    wid = cid * NS + sid

    pltpu.sync_copy(zrows, agg_sh.at[pl.ds(sid * SLAB, SLAB)])
    plsc.subcore_barrier()

    def step(c, carry):
        pltpu.sync_copy(src_r.at[wid, c], sidx)
        pltpu.async_copy(feat.at[sidx], rows, sem).wait()
        pltpu.sync_copy(dst_r.at[wid, c], didx)
        pltpu.sync_copy(rows, agg_sh.at[didx], add=True)
        return carry

    lax.fori_loop(0, NCH, step, 0)

    plsc.subcore_barrier()
    pltpu.sync_copy(agg_sh.at[pl.ds(sid * SLAB, SLAB)],
                    agg_out.at[cid, pl.ds(sid * SLAB, SLAB)])


def _make_sc_agg():
    mesh = plsc.VectorSubcoreMesh(core_axis_name="c", subcore_axis_name="s")
    return pl.kernel(
        _sc_agg_body,
        out_type=jax.ShapeDtypeStruct((NC, N_PAD, D), jnp.float32),
        mesh=mesh,
        scratch_types=[
            pltpu.VMEM((C,), jnp.int32),
            pltpu.VMEM((C,), jnp.int32),
            pltpu.VMEM((C, D), jnp.float32),
            pltpu.SemaphoreType.DMA,
            pltpu.VMEM_SHARED((N_PAD, D), jnp.float32),
        ],
    )


def _sc_deg_body(dst_r, zrows, ones_h, deg_out, didx, ones_v, deg_sh):
    cid = lax.axis_index("c")
    sid = lax.axis_index("s")
    wid = cid * NS + sid

    pltpu.sync_copy(zrows, deg_sh.at[pl.ds(sid * SLAB, SLAB)])
    pltpu.sync_copy(ones_h, ones_v)
    plsc.subcore_barrier()

    def step(c, carry):
        pltpu.sync_copy(dst_r.at[wid, c], didx)
        pltpu.sync_copy(ones_v, deg_sh.at[didx], add=True)
        return carry

    lax.fori_loop(0, NCH, step, 0)

    plsc.subcore_barrier()
    pltpu.sync_copy(deg_sh.at[pl.ds(sid * SLAB, SLAB)],
                    deg_out.at[cid, pl.ds(sid * SLAB, SLAB)])


def _make_sc_deg():
    mesh = plsc.VectorSubcoreMesh(core_axis_name="c", subcore_axis_name="s")
    return pl.kernel(
        _sc_deg_body,
        out_type=jax.ShapeDtypeStruct((NC, N_PAD, D), jnp.float32),
        mesh=mesh,
        scratch_types=[
            pltpu.VMEM((C,), jnp.int32),
            pltpu.VMEM((C, D), jnp.float32),
            pltpu.VMEM_SHARED((N_PAD, D), jnp.float32),
        ],
    )


def _tc_dense_body(relu, agg_ref, deg_ref, x_ref, wl_ref, b_ref, wr_ref, o_ref):
    agg = agg_ref[0] + agg_ref[1]
    deg = deg_ref[0, :, 0:1] + deg_ref[1, :, 0:1]
    mean = agg * (1.0 / jnp.maximum(deg, 1.0))
    h = (jnp.dot(mean, wl_ref[...], preferred_element_type=jnp.float32)
         + b_ref[...]
         + jnp.dot(x_ref[...], wr_ref[...], preferred_element_type=jnp.float32))
    o_ref[...] = jnp.maximum(h, 0.0) if relu else h


def _make_tc_dense(relu, br=512):
    grid = (N_PAD // br,)
    return pl.pallas_call(
        functools.partial(_tc_dense_body, relu),
        grid=grid,
        in_specs=[
            pl.BlockSpec((NC, br, D), lambda i: (0, i, 0)),
            pl.BlockSpec((NC, br, D), lambda i: (0, i, 0)),
            pl.BlockSpec((br, D), lambda i: (i, 0)),
            pl.BlockSpec((D, D), lambda i: (0, 0)),
            pl.BlockSpec((1, D), lambda i: (0, 0)),
            pl.BlockSpec((D, D), lambda i: (0, 0)),
        ],
        out_specs=pl.BlockSpec((br, D), lambda i: (i, 0)),
        out_shape=jax.ShapeDtypeStruct((N_PAD, D), jnp.float32),
    )


def kernel(x, edge_index, W1l, b1l, W1r, W2l, b2l, W2r):
    src = edge_index[0].astype(jnp.int32)
    dst = edge_index[1].astype(jnp.int32)
    pad = E_PAD - N_EDGES
    src_r = jnp.concatenate([src, jnp.zeros((pad,), jnp.int32)]).reshape(NW, NCH, C)
    dst_r = jnp.concatenate([dst, jnp.full((pad,), N_NODES, jnp.int32)]).reshape(NW, NCH, C)
    x_p = jnp.pad(x, ((0, N_PAD - N_NODES), (0, 0)))

    zrows = jnp.zeros((SLAB, D), jnp.float32)
    ones_h = jnp.ones((C, D), jnp.float32)

    sc_deg = _make_sc_deg()
    sc_agg = _make_sc_agg()
    tc1 = _make_tc_dense(True)
    tc2 = _make_tc_dense(False)

    deg_p = sc_deg(dst_r, zrows, ones_h)
    agg1 = sc_agg(x_p, src_r, dst_r, zrows)
    h = tc1(agg1, deg_p, x_p, W1l.T, b1l.reshape(1, D), W1r.T)
    agg2 = sc_agg(h, src_r, dst_r, zrows)
    out = tc2(agg2, deg_p, h, W2l.T, b2l.reshape(1, D), W2r.T)
    return out[:N_NODES]
